# baseline (speedup 1.0000x reference)
import jax
import jax.numpy as jnp
from jax.experimental import pallas as pl


def _relu_pallas(x):
    def body(x_ref, o_ref):
        o_ref[...] = jnp.maximum(x_ref[...], 0.0)

    return pl.pallas_call(
        body,
        out_shape=jax.ShapeDtypeStruct(x.shape, x.dtype),
    )(x)


def _lmul(ei, ew, xm):
    return jnp.zeros_like(xm).at[ei[1]].add(xm[ei[0]] * ew[:, None])


def _cheb(x, ei, ew, W, b):
    Bb, Fin, Vv = x.shape
    Kk = W.shape[0]
    x0 = jnp.transpose(x, (2, 0, 1)).reshape(Vv, Bb * Fin)
    xs = [x0]
    if Kk > 1:
        x1 = _lmul(ei, ew, x0)
        xs.append(x1)
        for _ in range(2, Kk):
            x2 = 2.0 * _lmul(ei, ew, x1) - x0
            xs.append(x2)
            x0, x1 = x1, x2
    xk = jnp.stack(xs, 0).reshape(Kk, Vv, Bb, Fin)
    return jnp.einsum('kvbi,kio->bov', xk, W) + b[None, :, None]


def _bn(x, g, b):
    m = x.mean(axis=(0, 2), keepdims=True)
    v = x.var(axis=(0, 2), keepdims=True)
    return g[None, :, None] * (x - m) / jnp.sqrt(v + 1e-5) + b[None, :, None]


def kernel(x, edge_index1, edge_weight1, edge_index0, edge_weight0, params):
    p = params
    r = jax.nn.relu
    ei1, ew1, ei0, ew0 = edge_index1, edge_weight1, edge_index0, edge_weight0
    h = r(_bn(_cheb(x, ei1, ew1, p['enc_c1_w'], p['enc_c1_b']), p['enc_bn1_g'], p['enc_bn1_b']))
    h = r(_bn(_cheb(h, ei1, ew1, p['enc_c2_w'], p['enc_c2_b']), p['enc_bn2_g'], p['enc_bn2_b']))
    ftr = h
    hp = h.reshape(h.shape[0], h.shape[1], -1, 4).max(axis=-1)
    d = r(_bn(_cheb(hp, ei0, ew0, p['dec0_c1_w'], p['dec0_c1_b']), p['dec0_bn1_g'], p['dec0_bn1_b']))
    d = r(_bn(_cheb(d, ei0, ew0, p['dec0_c2_w'], p['dec0_c2_b']), p['dec0_bn2_g'], p['dec0_bn2_b']))
    d = jnp.repeat(d, 4, axis=2)
    d = jnp.concatenate([d, ftr], axis=1)
    d = r(_bn(_cheb(d, ei1, ew1, p['decL_c1_w'], p['decL_c1_b']), p['decL_bn1_g'], p['decL_bn1_b']))
    d = r(_bn(_cheb(d, ei1, ew1, p['decL_c2_w'], p['decL_c2_b']), p['decL_bn2_g'], p['decL_bn2_b']))
    out = _cheb(d, ei1, ew1, p['head_w'], p['head_b'])
    return _relu_pallas(out)


# speedup vs baseline: 3.2470x; 3.2470x over previous
import functools

import jax
import jax.numpy as jnp
from jax import lax
from jax.experimental import pallas as pl
from jax.experimental.pallas import tpu as pltpu
from jax.experimental.pallas import tpu_sc as plsc

B = 2
NW = 32
NC = 2
V1, V0 = 50000, 12500
E1, E0 = V1 * 16, V0 * 16
R1, R0 = 1568, 392
V1P, V0P = NW * R1, NW * R0
EPAD = 256
EC = 128


def _make_lmul(Vpad, R, C):
  mesh = plsc.VectorSubcoreMesh(core_axis_name="c", subcore_axis_name="s")

  @functools.partial(
      pl.kernel,
      out_type=jax.ShapeDtypeStruct((Vpad, C), jnp.float32),
      mesh=mesh,
      compiler_params=pltpu.CompilerParams(
          needs_layout_passes=False, use_tc_tiling_on_sc=False),
      scratch_types=[
          pltpu.VMEM((EC,), jnp.int32),
          pltpu.VMEM((EC,), jnp.int32),
          pltpu.VMEM((EC,), jnp.float32),
          pltpu.VMEM((EC, C), jnp.float32),
          pltpu.VMEM((R, C), jnp.float32),
          pltpu.VMEM((48,), jnp.int32),
          pltpu.SemaphoreType.DMA,
      ],
  )
  def lmul(src_hbm, dst_hbm, w_hbm, off_hbm, x_hbm, out_hbm,
           srcv, dstv, wv, gv, accv, offv, sem):
    wid = lax.axis_index("s") * NC + lax.axis_index("c")
    pltpu.sync_copy(off_hbm, offv)
    lane0 = lax.iota(jnp.int32, 16)

    def off_at(pos):
      posv = jnp.full((16,), pos, jnp.int32)
      acc = None
      for g in range(3):
        vals = offv[pl.ds(g * 16, 16)]
        mv = jnp.where((lane0 + g * 16) == posv, vals,
                       jnp.int32(-2147483648))
        s = jnp.max(mv)
        acc = s if acc is None else jnp.maximum(acc, s)
      return acc

    e0 = off_at(wid)
    e1 = off_at(wid + 1)
    start = e0 - lax.rem(e0, 16)
    nch = lax.div(e1 - start + (EC - 1), EC)
    base_row = wid * R

    cols = [lax.iota(jnp.int32, 16) + c * 16 for c in range(C // 16)]

    def zero_body(r, _):
      for c in range(C // 16):
        accv[r, pl.ds(c * 16, 16)] = jnp.zeros((16,), jnp.float32)
      return 0
    lax.fori_loop(0, R, zero_body, 0, unroll=4)

    e0v = jnp.full((16,), e0, jnp.int32)
    e1v = jnp.full((16,), e1, jnp.int32)
    brv = jnp.full((16,), base_row, jnp.int32)
    lane = lax.iota(jnp.int32, 16)

    def chunk_body(i, _):
      eb = pl.multiple_of(start + i * EC, 16)
      pltpu.sync_copy(src_hbm.at[pl.ds(eb, EC)], srcv)
      pltpu.sync_copy(dst_hbm.at[pl.ds(eb, EC)], dstv)
      pltpu.sync_copy(w_hbm.at[pl.ds(eb, EC)], wv)
      pltpu.async_copy(x_hbm.at[srcv], gv, sem).wait()
      ebv = jnp.full((16,), eb, jnp.int32)
      for b in range(EC // 16):
        o16 = b * 16
        eid = ebv + (lane + o16)
        m = (eid >= e0v) & (eid < e1v)
        wvec = jnp.where(m, wv[pl.ds(o16, 16)], 0.0)
        dloc = dstv[pl.ds(o16, 16)] - brv
        dloc = jnp.minimum(jnp.maximum(dloc, 0), R - 1)
        for j in range(16):
          sp = jnp.full((16,), j, jnp.int32)
          wj = jnp.take_along_axis(wvec, sp, axis=0)
          dj = jnp.take_along_axis(dloc, sp, axis=0)
          for c in range(C // 16):
            row = gv[o16 + j, pl.ds(c * 16, 16)]
            plsc.addupdate_scatter(accv, [dj, cols[c]], row * wj)
      return 0

    lax.fori_loop(0, nch, chunk_body, 0)
    pltpu.sync_copy(accv, out_hbm.at[pl.ds(base_row, R)])

  return lmul


_LMULS = {}


def _lmul_sc(srcs, dsts, ws, offs, x, Vpad, R):
  C = x.shape[1]
  key = (Vpad, R, C)
  if key not in _LMULS:
    _LMULS[key] = _make_lmul(Vpad, R, C)
  return _LMULS[key](srcs, dsts, ws, offs, x)


def _combine(x0, x1, l1, Wc, bias, nvalid, with_stats, relu):
  rows, Fin = x0.shape
  Fout = Wc.shape[1]
  T = 1024 if rows % 1024 == 0 else 512
  grid = rows // T

  def body(x0r, x1r, l1r, wr, br, yr, sr=None):
    i = pl.program_id(0)
    cat = jnp.concatenate([x0r[...], x1r[...], l1r[...]], axis=1)
    y = jnp.dot(cat, wr[...], preferred_element_type=jnp.float32,
                precision=lax.Precision.HIGHEST)
    y = y + br[0:1, :]
    if relu:
      y = jnp.maximum(y, 0.0)
    yr[...] = y
    if with_stats:
      rowid = i * T + lax.broadcasted_iota(jnp.int32, (T, Fout), 0)
      ym = jnp.where(rowid < nvalid, y, 0.0)
      s1 = jnp.sum(ym, axis=0, keepdims=True)
      s2 = jnp.sum(ym * ym, axis=0, keepdims=True)

      @pl.when(i == 0)
      def _():
        sr[...] = jnp.zeros_like(sr)

      sr[0:1, :] += s1
      sr[1:2, :] += s2

  out_shape = [jax.ShapeDtypeStruct((rows, Fout), jnp.float32)]
  out_specs = [pl.BlockSpec((T, Fout), lambda i: (i, 0))]
  if with_stats:
    out_shape.append(jax.ShapeDtypeStruct((8, Fout), jnp.float32))
    out_specs.append(pl.BlockSpec((8, Fout), lambda i: (0, 0)))

  res = pl.pallas_call(
      body,
      grid=(grid,),
      in_specs=[
          pl.BlockSpec((T, Fin), lambda i: (i, 0)),
          pl.BlockSpec((T, Fin), lambda i: (i, 0)),
          pl.BlockSpec((T, Fin), lambda i: (i, 0)),
          pl.BlockSpec(Wc.shape, lambda i: (0, 0)),
          pl.BlockSpec((8, Fout), lambda i: (0, 0)),
      ],
      out_specs=out_specs,
      out_shape=out_shape,
  )(x0, x1, l1, Wc, bias)
  return res if with_stats else res[0]


def _norm_relu(y, stats, gb, nvalid):
  rows, Fout = y.shape
  T = 1024 if rows % 1024 == 0 else 512

  def body(yr, sr, gr, outr):
    cnt = jnp.float32(nvalid)
    s1 = sr[0:1, :] / cnt
    var = sr[1:2, :] / cnt - s1 * s1
    scale = gr[0:1, :] * lax.rsqrt(var + 1e-5)
    shift = gr[1:2, :] - s1 * scale
    outr[...] = jnp.maximum(yr[...] * scale + shift, 0.0)

  return pl.pallas_call(
      body,
      grid=(rows // T,),
      in_specs=[
          pl.BlockSpec((T, Fout), lambda i: (i, 0)),
          pl.BlockSpec((8, Fout), lambda i: (0, 0)),
          pl.BlockSpec((8, Fout), lambda i: (0, 0)),
      ],
      out_specs=pl.BlockSpec((T, Fout), lambda i: (i, 0)),
      out_shape=jax.ShapeDtypeStruct((rows, Fout), jnp.float32),
  )(y, stats, gb)


def _pool4(h, Vout_pad):
  C = h.shape[1]
  T = 448 if Vout_pad % 448 == 0 else 128

  def body(hr, outr):
    x = hr[...]
    x = x.reshape(T, 4, C)
    outr[...] = jnp.max(x, axis=1)

  return pl.pallas_call(
      body,
      grid=(Vout_pad // T,),
      in_specs=[pl.BlockSpec((4 * T, C), lambda i: (i, 0))],
      out_specs=pl.BlockSpec((T, C), lambda i: (i, 0)),
      out_shape=jax.ShapeDtypeStruct((Vout_pad, C), jnp.float32),
  )(h)


def _unpool_concat(d, ftr, Vout_pad):
  F1 = d.shape[1] // B
  F2 = ftr.shape[1] // B
  Co = B * (F1 + F2)
  T = 512

  def body(dr, fr, outr):
    dd = dr[...]
    up = jnp.broadcast_to(dd[:, None, :], (T // 4, 4, B * F1)).reshape(T, B * F1)
    ff = fr[...]
    for b in range(B):
      outr[:, b * (F1 + F2):b * (F1 + F2) + F1] = up[:, b * F1:(b + 1) * F1]
      outr[:, b * (F1 + F2) + F1:(b + 1) * (F1 + F2)] = ff[:, b * F2:(b + 1) * F2]

  return pl.pallas_call(
      body,
      grid=(Vout_pad // T,),
      in_specs=[
          pl.BlockSpec((T // 4, B * F1), lambda i: (i, 0)),
          pl.BlockSpec((T, B * F2), lambda i: (i, 0)),
      ],
      out_specs=pl.BlockSpec((T, Co), lambda i: (i, 0)),
      out_shape=jax.ShapeDtypeStruct((Vout_pad, Co), jnp.float32),
  )(d, ftr)


def _prep_edges(ei, ew, R):
  src, dst = ei[0], ei[1]
  E_ = dst.shape[0]
  bucket = jnp.minimum(dst // R, NW - 1)
  scramble = (jnp.arange(E_, dtype=jnp.uint32) * jnp.uint32(2654435761)) % jnp.uint32(E_)
  key = bucket.astype(jnp.uint32) * jnp.uint32(E_) + scramble
  order = jnp.argsort(key)
  srcs = src[order]
  dsts = dst[order]
  ws = ew[order]
  E = src.shape[0]
  counts = jnp.bincount(bucket, length=NW)
  offs = jnp.concatenate([
      jnp.zeros((1,), jnp.int32),
      jnp.cumsum(counts).astype(jnp.int32), jnp.zeros((15,), jnp.int32)])
  srcs = jnp.pad(srcs, (0, EPAD))
  dsts = jnp.pad(dsts, (0, EPAD))
  ws = jnp.pad(ws, (0, EPAD))
  return srcs, dsts, ws, offs


def _prep_w(W):
  K, Fin, Fout = W.shape
  Wm = jnp.concatenate([W[0] - W[2], W[1], 2.0 * W[2]], axis=0)
  return Wm


def _prep_b(b):
  return jnp.broadcast_to(b[None, :], (8, b.shape[0]))


def _prep_gb(g, b):
  z = jnp.zeros((6, g.shape[0]), jnp.float32)
  return jnp.concatenate([g[None, :], b[None, :], z], axis=0)


def _cheb_layer(x, edges, Vpad, R, nvalid_rows, Wm, bias, gb=None, head=False):
  srcs, dsts, ws, offs = edges
  C = x.shape[1]
  Fin = C // B
  x1 = _lmul_sc(srcs, dsts, ws, offs, x, Vpad, R)
  l1 = _lmul_sc(srcs, dsts, ws, offs, x1, Vpad, R)
  rows = Vpad * B
  x0r = x.reshape(rows, Fin)
  x1r = x1.reshape(rows, Fin)
  l1r = l1.reshape(rows, Fin)
  if head:
    y = _combine(x0r, x1r, l1r, Wm, bias, nvalid_rows, False, True)
    return y
  y, st = _combine(x0r, x1r, l1r, Wm, bias, nvalid_rows, True, False)
  h = _norm_relu(y, st, gb, nvalid_rows)
  return h


def kernel(x, edge_index1, edge_weight1, edge_index0, edge_weight0, params):
  p = params
  e1 = _prep_edges(edge_index1, edge_weight1, R1)
  e0 = _prep_edges(edge_index0, edge_weight0, R0)

  x0 = jnp.transpose(x, (2, 0, 1)).reshape(V1, B * x.shape[1])
  x0 = jnp.pad(x0, ((0, V1P - V1), (0, 0)))

  nv1 = V1 * B
  nv0 = V0 * B

  h = _cheb_layer(x0, e1, V1P, R1, nv1, _prep_w(p['enc_c1_w']),
                  _prep_b(p['enc_c1_b']), _prep_gb(p['enc_bn1_g'], p['enc_bn1_b']))
  h = h.reshape(V1P, -1)
  h = _cheb_layer(h, e1, V1P, R1, nv1, _prep_w(p['enc_c2_w']),
                  _prep_b(p['enc_c2_b']), _prep_gb(p['enc_bn2_g'], p['enc_bn2_b']))
  ftr = h.reshape(V1P, -1)

  hp = _pool4(ftr, V0P)
  d = _cheb_layer(hp, e0, V0P, R0, nv0, _prep_w(p['dec0_c1_w']),
                  _prep_b(p['dec0_c1_b']), _prep_gb(p['dec0_bn1_g'], p['dec0_bn1_b']))
  d = d.reshape(V0P, -1)
  d = _cheb_layer(d, e0, V0P, R0, nv0, _prep_w(p['dec0_c2_w']),
                  _prep_b(p['dec0_c2_b']), _prep_gb(p['dec0_bn2_g'], p['dec0_bn2_b']))
  d = d.reshape(V0P, -1)

  u = _unpool_concat(d, ftr, V1P)
  d = _cheb_layer(u, e1, V1P, R1, nv1, _prep_w(p['decL_c1_w']),
                  _prep_b(p['decL_c1_b']), _prep_gb(p['decL_bn1_g'], p['decL_bn1_b']))
  d = d.reshape(V1P, -1)
  d = _cheb_layer(d, e1, V1P, R1, nv1, _prep_w(p['decL_c2_w']),
                  _prep_b(p['decL_c2_b']), _prep_gb(p['decL_bn2_g'], p['decL_bn2_b']))
  d = d.reshape(V1P, -1)

  out = _cheb_layer(d, e1, V1P, R1, nv1, _prep_w(p['head_w']),
                    _prep_b(p['head_b']), head=True)
  out = out[:V1 * B].reshape(V1, B, -1)
  return jnp.transpose(out, (1, 2, 0))


# speedup vs baseline: 5.8953x; 1.8156x over previous
import functools

import jax
import jax.numpy as jnp
from jax import lax
from jax.experimental import pallas as pl
from jax.experimental.pallas import tpu as pltpu
from jax.experimental.pallas import tpu_sc as plsc

B = 2
NW = 32
NC = 2
V1, V0 = 50000, 12500
E1, E0 = V1 * 16, V0 * 16
R1, R0 = 1568, 392
V1P, V0P = NW * R1, NW * R0
EPAD = 2304


def _make_lmul(Vpad, R, C):
  mesh = plsc.VectorSubcoreMesh(core_axis_name="c", subcore_axis_name="s")
  EC = 128 if C == 64 else 256

  @functools.partial(
      pl.kernel,
      out_type=jax.ShapeDtypeStruct((Vpad, C), jnp.float32),
      mesh=mesh,
      compiler_params=pltpu.CompilerParams(
          needs_layout_passes=False, use_tc_tiling_on_sc=False),
      scratch_types=[
          pltpu.VMEM((3, EC), jnp.int32),
          pltpu.VMEM((3, EC), jnp.int32),
          pltpu.VMEM((3, EC), jnp.float32),
          pltpu.VMEM((2, EC, C), jnp.float32),
          pltpu.VMEM((R, C), jnp.float32),
          pltpu.VMEM((48,), jnp.int32),
          pltpu.SemaphoreType.DMA,
          pltpu.SemaphoreType.DMA,
          pltpu.SemaphoreType.DMA,
          pltpu.SemaphoreType.DMA,
          pltpu.SemaphoreType.DMA,
      ],
  )
  def lmul(src_hbm, dst_hbm, w_hbm, off_hbm, x_hbm, out_hbm,
           srcv, dstv, wv, gv, accv, offv, se0, se1, se2, sg0, sg1):
    se = (se0, se1, se2)
    sg = (sg0, sg1)
    wid = lax.axis_index("s") * NC + lax.axis_index("c")
    pltpu.sync_copy(off_hbm, offv)
    lane = lax.iota(jnp.int32, 16)

    def off_at(pos):
      posv = jnp.full((16,), pos, jnp.int32)
      acc = None
      for g in range(3):
        vals = offv[pl.ds(g * 16, 16)]
        mv = jnp.where((lane + g * 16) == posv, vals,
                       jnp.int32(-2147483648))
        s = jnp.max(mv)
        acc = s if acc is None else jnp.maximum(acc, s)
      return acc

    e0 = off_at(wid)
    e1 = off_at(wid + 1)
    start = e0 - lax.rem(e0, 16)
    nch = lax.div(e1 - start + (EC - 1), EC)
    ngroups = lax.div(nch + 5, 6)
    base_row = wid * R

    cols = [lax.iota(jnp.int32, 16) + c * 16 for c in range(C // 16)]

    def eb_of(i):
      return pl.multiple_of(start + i * EC, 16)

    def e_descs(i, b):
      eb = eb_of(i)
      return (
          (src_hbm.at[pl.ds(eb, EC)], srcv.at[b], se[b]),
          (dst_hbm.at[pl.ds(eb, EC)], dstv.at[b], se[b]),
          (w_hbm.at[pl.ds(eb, EC)], wv.at[b], se[b]),
      )

    def start_e(i, b):
      for s_, d_, m_ in e_descs(i, b):
        pltpu.async_copy(s_, d_, m_)

    def wait_e(i, b):
      for s_, d_, m_ in e_descs(i, b):
        pltpu.make_async_copy(s_, d_, m_).wait()

    def start_g(be, bg):
      pltpu.async_copy(x_hbm.at[srcv.at[be]], gv.at[bg], sg[bg])

    def wait_g(be, bg):
      pltpu.make_async_copy(x_hbm.at[srcv.at[be]], gv.at[bg], sg[bg]).wait()

    def zero_body(r, _):
      for c in range(C // 16):
        accv[r, pl.ds(c * 16, 16)] = jnp.zeros((16,), jnp.float32)
      return 0

    start_e(0, 0)
    start_e(1, 1)
    lax.fori_loop(0, R, zero_body, 0, unroll=4)
    wait_e(0, 0)
    start_g(0, 0)

    e0v = jnp.full((16,), e0, jnp.int32)
    e1v = jnp.full((16,), e1, jnp.int32)
    brv = jnp.full((16,), base_row, jnp.int32)

    def compute(i, be, bg):
      eb = start + i * EC

      def batch(bi, _):
        o16 = bi * 16
        ebv = jnp.full((16,), eb + o16, jnp.int32)
        eid = ebv + lane
        m = (eid >= e0v) & (eid < e1v)
        wvec = jnp.where(m, wv[be, pl.ds(o16, 16)], 0.0)
        dloc = dstv[be, pl.ds(o16, 16)] - brv
        dloc = jnp.minimum(jnp.maximum(dloc, 0), R - 1)
        for j in range(16):
          sp = jnp.full((16,), j, jnp.int32)
          wj = jnp.take_along_axis(wvec, sp, axis=0)
          dj = jnp.take_along_axis(dloc, sp, axis=0)
          for c in range(C // 16):
            row = gv[bg, o16 + j, pl.ds(c * 16, 16)]
            plsc.addupdate_scatter(accv, [dj, cols[c]], row * wj)
        return 0

      lax.fori_loop(0, EC // 16, batch, 0)

    def group_body(g, _):
      i0 = g * 6
      for k in range(6):
        i = i0 + k
        be, bg = k % 3, k % 2
        start_e(i + 2, (k + 2) % 3)
        wait_e(i + 1, (k + 1) % 3)
        start_g((k + 1) % 3, (k + 1) % 2)
        wait_g(be, bg)
        compute(i, be, bg)
      return 0

    lax.fori_loop(0, ngroups, group_body, 0)

    n6 = ngroups * 6
    wait_e(n6 + 1, 1)
    wait_g(0, 0)
    pltpu.sync_copy(accv, out_hbm.at[pl.ds(base_row, R)])

  return lmul


_LMULS = {}


def _lmul_sc(srcs, dsts, ws, offs, x, Vpad, R):
  C = x.shape[1]
  key = (Vpad, R, C)
  if key not in _LMULS:
    _LMULS[key] = _make_lmul(Vpad, R, C)
  return _LMULS[key](srcs, dsts, ws, offs, x)


def _combine(x0, x1, l1, Wc, bias, nvalid, with_stats, relu):
  rows, Fin = x0.shape
  Fout = Wc.shape[1]
  T = 1024 if rows % 1024 == 0 else 512
  grid = rows // T

  def body(x0r, x1r, l1r, wr, br, yr, sr=None):
    i = pl.program_id(0)
    cat = jnp.concatenate([x0r[...], x1r[...], l1r[...]], axis=1)
    y = jnp.dot(cat, wr[...], preferred_element_type=jnp.float32,
                precision=lax.Precision.HIGHEST)
    y = y + br[0:1, :]
    if relu:
      y = jnp.maximum(y, 0.0)
    yr[...] = y
    if with_stats:
      rowid = i * T + lax.broadcasted_iota(jnp.int32, (T, Fout), 0)
      ym = jnp.where(rowid < nvalid, y, 0.0)
      s1 = jnp.sum(ym, axis=0, keepdims=True)
      s2 = jnp.sum(ym * ym, axis=0, keepdims=True)

      @pl.when(i == 0)
      def _():
        sr[...] = jnp.zeros_like(sr)

      sr[0:1, :] += s1
      sr[1:2, :] += s2

  out_shape = [jax.ShapeDtypeStruct((rows, Fout), jnp.float32)]
  out_specs = [pl.BlockSpec((T, Fout), lambda i: (i, 0))]
  if with_stats:
    out_shape.append(jax.ShapeDtypeStruct((8, Fout), jnp.float32))
    out_specs.append(pl.BlockSpec((8, Fout), lambda i: (0, 0)))

  res = pl.pallas_call(
      body,
      grid=(grid,),
      in_specs=[
          pl.BlockSpec((T, Fin), lambda i: (i, 0)),
          pl.BlockSpec((T, Fin), lambda i: (i, 0)),
          pl.BlockSpec((T, Fin), lambda i: (i, 0)),
          pl.BlockSpec(Wc.shape, lambda i: (0, 0)),
          pl.BlockSpec((8, Fout), lambda i: (0, 0)),
      ],
      out_specs=out_specs,
      out_shape=out_shape,
  )(x0, x1, l1, Wc, bias)
  return res if with_stats else res[0]


def _norm_relu(y, stats, gb, nvalid):
  rows, Fout = y.shape
  T = 1024 if rows % 1024 == 0 else 512

  def body(yr, sr, gr, outr):
    cnt = jnp.float32(nvalid)
    s1 = sr[0:1, :] / cnt
    var = sr[1:2, :] / cnt - s1 * s1
    scale = gr[0:1, :] * lax.rsqrt(var + 1e-5)
    shift = gr[1:2, :] - s1 * scale
    outr[...] = jnp.maximum(yr[...] * scale + shift, 0.0)

  return pl.pallas_call(
      body,
      grid=(rows // T,),
      in_specs=[
          pl.BlockSpec((T, Fout), lambda i: (i, 0)),
          pl.BlockSpec((8, Fout), lambda i: (0, 0)),
          pl.BlockSpec((8, Fout), lambda i: (0, 0)),
      ],
      out_specs=pl.BlockSpec((T, Fout), lambda i: (i, 0)),
      out_shape=jax.ShapeDtypeStruct((rows, Fout), jnp.float32),
  )(y, stats, gb)


def _pool4(h, Vout_pad):
  C = h.shape[1]
  T = 448 if Vout_pad % 448 == 0 else 128

  def body(hr, outr):
    x = hr[...]
    x = x.reshape(T, 4, C)
    outr[...] = jnp.max(x, axis=1)

  return pl.pallas_call(
      body,
      grid=(Vout_pad // T,),
      in_specs=[pl.BlockSpec((4 * T, C), lambda i: (i, 0))],
      out_specs=pl.BlockSpec((T, C), lambda i: (i, 0)),
      out_shape=jax.ShapeDtypeStruct((Vout_pad, C), jnp.float32),
  )(h)


def _unpool_concat(d, ftr, Vout_pad):
  F1 = d.shape[1] // B
  F2 = ftr.shape[1] // B
  Co = B * (F1 + F2)
  T = 512

  def body(dr, fr, outr):
    dd = dr[...]
    up = jnp.broadcast_to(dd[:, None, :], (T // 4, 4, B * F1)).reshape(T, B * F1)
    ff = fr[...]
    for b in range(B):
      outr[:, b * (F1 + F2):b * (F1 + F2) + F1] = up[:, b * F1:(b + 1) * F1]
      outr[:, b * (F1 + F2) + F1:(b + 1) * (F1 + F2)] = ff[:, b * F2:(b + 1) * F2]

  return pl.pallas_call(
      body,
      grid=(Vout_pad // T,),
      in_specs=[
          pl.BlockSpec((T // 4, B * F1), lambda i: (i, 0)),
          pl.BlockSpec((T, B * F2), lambda i: (i, 0)),
      ],
      out_specs=pl.BlockSpec((T, Co), lambda i: (i, 0)),
      out_shape=jax.ShapeDtypeStruct((Vout_pad, Co), jnp.float32),
  )(d, ftr)


def _prep_edges(ei, ew, R):
  src, dst = ei[0], ei[1]
  E_ = dst.shape[0]
  bucket = jnp.minimum(dst // R, NW - 1)
  scramble = (jnp.arange(E_, dtype=jnp.uint32) * jnp.uint32(2654435761)) % jnp.uint32(E_)
  key = bucket.astype(jnp.uint32) * jnp.uint32(E_) + scramble
  order = jnp.argsort(key)
  srcs = src[order]
  dsts = dst[order]
  ws = ew[order]
  E = src.shape[0]
  counts = jnp.bincount(bucket, length=NW)
  offs = jnp.concatenate([
      jnp.zeros((1,), jnp.int32),
      jnp.cumsum(counts).astype(jnp.int32), jnp.zeros((15,), jnp.int32)])
  srcs = jnp.pad(srcs, (0, EPAD))
  dsts = jnp.pad(dsts, (0, EPAD))
  ws = jnp.pad(ws, (0, EPAD))
  return srcs, dsts, ws, offs


def _prep_w(W):
  K, Fin, Fout = W.shape
  Wm = jnp.concatenate([W[0] - W[2], W[1], 2.0 * W[2]], axis=0)
  return Wm


def _prep_b(b):
  return jnp.broadcast_to(b[None, :], (8, b.shape[0]))


def _prep_gb(g, b):
  z = jnp.zeros((6, g.shape[0]), jnp.float32)
  return jnp.concatenate([g[None, :], b[None, :], z], axis=0)


def _cheb_layer(x, edges, Vpad, R, nvalid_rows, Wm, bias, gb=None, head=False):
  srcs, dsts, ws, offs = edges
  C = x.shape[1]
  Fin = C // B
  x1 = _lmul_sc(srcs, dsts, ws, offs, x, Vpad, R)
  l1 = _lmul_sc(srcs, dsts, ws, offs, x1, Vpad, R)
  rows = Vpad * B
  x0r = x.reshape(rows, Fin)
  x1r = x1.reshape(rows, Fin)
  l1r = l1.reshape(rows, Fin)
  if head:
    y = _combine(x0r, x1r, l1r, Wm, bias, nvalid_rows, False, True)
    return y
  y, st = _combine(x0r, x1r, l1r, Wm, bias, nvalid_rows, True, False)
  h = _norm_relu(y, st, gb, nvalid_rows)
  return h


def kernel(x, edge_index1, edge_weight1, edge_index0, edge_weight0, params):
  p = params
  e1 = _prep_edges(edge_index1, edge_weight1, R1)
  e0 = _prep_edges(edge_index0, edge_weight0, R0)

  x0 = jnp.transpose(x, (2, 0, 1)).reshape(V1, B * x.shape[1])
  x0 = jnp.pad(x0, ((0, V1P - V1), (0, 0)))

  nv1 = V1 * B
  nv0 = V0 * B

  h = _cheb_layer(x0, e1, V1P, R1, nv1, _prep_w(p['enc_c1_w']),
                  _prep_b(p['enc_c1_b']), _prep_gb(p['enc_bn1_g'], p['enc_bn1_b']))
  h = h.reshape(V1P, -1)
  h = _cheb_layer(h, e1, V1P, R1, nv1, _prep_w(p['enc_c2_w']),
                  _prep_b(p['enc_c2_b']), _prep_gb(p['enc_bn2_g'], p['enc_bn2_b']))
  ftr = h.reshape(V1P, -1)

  hp = _pool4(ftr, V0P)
  d = _cheb_layer(hp, e0, V0P, R0, nv0, _prep_w(p['dec0_c1_w']),
                  _prep_b(p['dec0_c1_b']), _prep_gb(p['dec0_bn1_g'], p['dec0_bn1_b']))
  d = d.reshape(V0P, -1)
  d = _cheb_layer(d, e0, V0P, R0, nv0, _prep_w(p['dec0_c2_w']),
                  _prep_b(p['dec0_c2_b']), _prep_gb(p['dec0_bn2_g'], p['dec0_bn2_b']))
  d = d.reshape(V0P, -1)

  u = _unpool_concat(d, ftr, V1P)
  d = _cheb_layer(u, e1, V1P, R1, nv1, _prep_w(p['decL_c1_w']),
                  _prep_b(p['decL_c1_b']), _prep_gb(p['decL_bn1_g'], p['decL_bn1_b']))
  d = d.reshape(V1P, -1)
  d = _cheb_layer(d, e1, V1P, R1, nv1, _prep_w(p['decL_c2_w']),
                  _prep_b(p['decL_c2_b']), _prep_gb(p['decL_bn2_g'], p['decL_bn2_b']))
  d = d.reshape(V1P, -1)

  out = _cheb_layer(d, e1, V1P, R1, nv1, _prep_w(p['head_w']),
                    _prep_b(p['head_b']), head=True)
  out = out[:V1 * B].reshape(V1, B, -1)
  return jnp.transpose(out, (1, 2, 0))
